# Initial kernel scaffold; baseline (speedup 1.0000x reference)
import jax
import jax.numpy as jnp
from jax.experimental import pallas as pl


def kernel(inputs, weight, k, sorted_expert_idxs, sorted_scattered_idxs, expert_offsets, gates):
    raise NotImplementedError("write your pallas kernel here")


# speedup vs baseline: 235.3080x; 235.3080x over previous
import functools

import jax
import jax.numpy as jnp
from jax import lax
from jax.experimental import pallas as pl
from jax.experimental.pallas import tpu as pltpu
from jax.experimental.pallas import tpu_sc as plsc


def _gather_rows_sc(table, idx):
    n_rows, d = table.shape
    s = idx.shape[0]
    info = plsc.get_sparse_core_info()
    nc, ns = info.num_cores, info.num_subcores
    nw = nc * ns
    assert s % nw == 0 and d % info.num_lanes == 0
    b_per_w = s // nw
    mesh = plsc.VectorSubcoreMesh(core_axis_name="c", subcore_axis_name="s")

    @functools.partial(
        pl.kernel,
        out_type=jax.ShapeDtypeStruct((s, d), table.dtype),
        mesh=mesh,
        scratch_types=[
            pltpu.VMEM((b_per_w,), jnp.int32),
            pltpu.VMEM((b_per_w, d), table.dtype),
            pltpu.SemaphoreType.DMA,
        ],
    )
    def gather_kernel(table_hbm, idx_hbm, out_hbm, idx_v, rows_v, sem):
        wid = lax.axis_index("s") * nc + lax.axis_index("c")
        base = wid * b_per_w
        pltpu.sync_copy(idx_hbm.at[pl.ds(base, b_per_w)], idx_v)
        pltpu.async_copy(table_hbm.at[idx_v], rows_v, sem).wait()
        pltpu.sync_copy(rows_v, out_hbm.at[pl.ds(base, b_per_w)])

    return gather_kernel(table, idx)


def _work_list(sei, n_experts, blk):
    s = sei.shape[0]
    nb = s // blk
    nw = nb + n_experts - 1
    blocks = sei.reshape(nb, blk)
    first = blocks[:, 0].astype(jnp.int32)
    last = blocks[:, -1].astype(jnp.int32)
    counts = last - first + 1
    cum = jnp.cumsum(counts)
    total = cum[-1]
    cumstart = cum - counts
    w_ids = jnp.arange(nw, dtype=jnp.int32)
    blk_of_w = jnp.minimum(
        jnp.searchsorted(cum, w_ids, side="right").astype(jnp.int32), nb - 1
    )
    expert_w = first[blk_of_w] + (w_ids - cumstart[blk_of_w])
    valid_w = (w_ids < total).astype(jnp.int32)
    expert_w = jnp.where(valid_w == 1, expert_w, last[nb - 1]).astype(jnp.int32)
    first_w = ((w_ids == cumstart[blk_of_w]) & (w_ids < total)).astype(jnp.int32)
    return blk_of_w, expert_w, first_w, valid_w, nw


def _grouped_matmul_body(blk_ref, exp_ref, fst_ref, vld_ref,
                         x_ref, w_ref, sei_ref, out_ref):
    w = pl.program_id(0)
    e = exp_ref[w]
    mask = ((sei_ref[...] == e) & (vld_ref[w] == 1)).astype(jnp.float32)
    xm = x_ref[...] * mask
    contrib = lax.dot_general(
        xm, w_ref[0], (((1,), (1,)), ((), ())),
        preferred_element_type=jnp.float32,
    )

    @pl.when(fst_ref[w] == 1)
    def _():
        out_ref[...] = contrib

    @pl.when(fst_ref[w] == 0)
    def _():
        out_ref[...] += contrib


def _grouped_matmul(xg, weight, sei, blk=256, interpret=False):
    s, d_in = xg.shape
    n_experts, d_out, _ = weight.shape
    blk_w, exp_w, fst_w, vld_w, nw = _work_list(sei, n_experts, blk)
    sei2d = sei.reshape(s, 1)
    grid_spec = pltpu.PrefetchScalarGridSpec(
        num_scalar_prefetch=4,
        grid=(nw,),
        in_specs=[
            pl.BlockSpec((blk, d_in), lambda w, b, e, f, v: (b[w], 0)),
            pl.BlockSpec((1, d_out, d_in), lambda w, b, e, f, v: (e[w], 0, 0)),
            pl.BlockSpec((blk, 1), lambda w, b, e, f, v: (b[w], 0)),
        ],
        out_specs=pl.BlockSpec((blk, d_out), lambda w, b, e, f, v: (b[w], 0)),
    )
    return pl.pallas_call(
        _grouped_matmul_body,
        grid_spec=grid_spec,
        out_shape=jax.ShapeDtypeStruct((s, d_out), jnp.float32),
        compiler_params=pltpu.CompilerParams(
            dimension_semantics=("arbitrary",)),
        interpret=interpret,
    )(blk_w, exp_w, fst_w, vld_w, xg, weight, sei2d)


def _combine(out_s, gates, d_out, interpret=False):
    t, kk = gates.shape
    y = out_s.reshape(t, kk * d_out)
    bt = min(t, 512)

    def body(g_ref, y_ref, o_ref):
        g = g_ref[...]
        yv = y_ref[...]
        acc = g[:, 0:1] * yv[:, 0:d_out]
        for j in range(1, kk):
            acc = acc + g[:, j:j + 1] * yv[:, j * d_out:(j + 1) * d_out]
        o_ref[...] = acc

    return pl.pallas_call(
        body,
        grid=(t // bt,),
        in_specs=[
            pl.BlockSpec((bt, kk), lambda i: (i, 0)),
            pl.BlockSpec((bt, kk * d_out), lambda i: (i, 0)),
        ],
        out_specs=pl.BlockSpec((bt, d_out), lambda i: (i, 0)),
        out_shape=jax.ShapeDtypeStruct((t, d_out), jnp.float32),
        interpret=interpret,
    )(gates, y)


def kernel(inputs, weight, k, sorted_expert_idxs, sorted_scattered_idxs,
           expert_offsets, gates):
    del k, expert_offsets
    kk = gates.shape[1]
    d_out = weight.shape[1]
    token_idxs = (sorted_scattered_idxs // kk).astype(jnp.int32)
    xg = _gather_rows_sc(inputs, token_idxs)
    out_s = _grouped_matmul(xg, weight, sorted_expert_idxs)
    return _combine(out_s, gates, d_out)
